# baseline (speedup 1.0000x reference)
import functools

import jax
import jax.numpy as jnp
from jax import lax
from jax.experimental import pallas as pl
from jax.experimental.pallas import tpu as pltpu
from jax.experimental.pallas import tpu_sc as plsc

N = 10000
E = 320000
D = 128
DH = 64
NP = 10240
TPB = NP // 16
CH = 128
NCHUNK = 160
EPT = NCHUNK * CH
E_PAD = 16 * EPT
MM_BLK = 512


def _mm_body(x_ref, w_ref, o_ref):
    o_ref[0] = jnp.dot(x_ref[...], w_ref[0],
                       preferred_element_type=jnp.float32)


def _matmul(x_pad, w_cat):
    return pl.pallas_call(
        _mm_body,
        out_shape=jax.ShapeDtypeStruct((2, NP, DH), jnp.float32),
        grid=(NP // MM_BLK, 2),
        in_specs=[
            pl.BlockSpec((MM_BLK, D), lambda i, j: (i, 0)),
            pl.BlockSpec((1, D, DH), lambda i, j: (j, 0, 0)),
        ],
        out_specs=pl.BlockSpec((1, MM_BLK, DH), lambda i, j: (j, i, 0)),
    )(x_pad, w_cat)


def _splat(vec_ref, i):
    return plsc.load_gather(vec_ref, [jnp.full((16,), i, jnp.int32)])


SLOTS = 5
LEAD = 2


def _sc_body(h_hbm, src_hbm, dst_hbm, b_hbm, out0, out1, gout,
             acc_sh, deg_sh, srcv, dstv,
             buf0, buf1, buf2, buf3, buf4,
             disv, onesv, bv,
             semg0, semg1, semg2, semg3, semg4,
             sems0, sems1, sems2, sems3, sems4, semh):
    cid = lax.axis_index("c")
    sid = lax.axis_index("s")

    bufs = (buf0, buf1, buf2, buf3, buf4)
    gsems = (semg0, semg1, semg2, semg3, semg4)
    ssems = (sems0, sems1, sems2, sems3, sems4)

    zeros16 = jnp.zeros((16,), jnp.float32)
    ones16 = jnp.ones((16,), jnp.float32)

    pltpu.sync_copy(src_hbm.at[sid], srcv)
    pltpu.sync_copy(dst_hbm.at[sid], dstv)
    pltpu.sync_copy(b_hbm.at[pl.ds(cid * DH, DH)], bv)

    @pl.when(cid == 1)
    def _():
        def _off(j, _):
            for v in range(CH // 16):
                sl = pl.ds(16 * v, 16)
                srcv[j, sl] = srcv[j, sl] + NP
            return 0
        lax.fori_loop(0, NCHUNK, _off, 0)

    def _zrow(r, _):
        for j in range(4):
            buf0[r, pl.ds(16 * j, 16)] = zeros16
        return 0
    lax.fori_loop(0, CH, _zrow, 0)

    def _zvec(k, _):
        disv[pl.ds(k * 16, 16)] = zeros16
        return 0
    lax.fori_loop(0, TPB // 16, _zvec, 0)

    for j in range(CH // 16):
        onesv[pl.ds(16 * j, 16)] = ones16

    for k in range(TPB // CH):
        pltpu.sync_copy(buf0, acc_sh.at[pl.ds(sid * TPB + k * CH, CH), :])
    pltpu.sync_copy(disv, deg_sh.at[pl.ds(sid * TPB, TPB)])

    plsc.subcore_barrier()

    def _hist(p, _):
        for i in range(8):
            pltpu.async_copy(onesv, deg_sh.at[dstv.at[p * 8 + i]], semh,
                             add=True)
        for i in range(8):
            pltpu.make_async_copy(onesv, deg_sh.at[dstv.at[p * 8 + i]],
                                  semh).wait()
        return 0
    lax.fori_loop(0, NCHUNK // 8, _hist, 0)

    plsc.subcore_barrier()

    pltpu.sync_copy(deg_sh.at[pl.ds(sid * TPB, TPB)], disv)

    def _newton(k, _):
        d = disv[pl.ds(k * 16, 16)] + 1.0
        y = 0.5 * (d + 1.0)
        for _i in range(16):
            y = 0.5 * (y + d / y)
        disv[pl.ds(k * 16, 16)] = 1.0 / y
        return 0
    lax.fori_loop(0, TPB // 16, _newton, 0)

    for k in range(TPB // CH):
        row0 = sid * TPB + k * CH
        pltpu.sync_copy(h_hbm.at[cid, pl.ds(row0, CH), :], buf0)

        def _scale(q, _):
            for u in range(4):
                r = q * 4 + u
                dsp = _splat(disv, k * CH + r)
                for j in range(4):
                    sl = pl.ds(16 * j, 16)
                    buf0[r, sl] = buf0[r, sl] * dsp
            return 0
        lax.fori_loop(0, CH // 4, _scale, 0)
        pltpu.sync_copy(buf0, gout.at[pl.ds(cid * NP + row0, CH), :])

    plsc.subcore_barrier()

    def _gfire(j, s):
        pltpu.async_copy(gout.at[srcv.at[j]], bufs[s], gsems[s])

    def _gwait(s):
        pltpu.make_async_copy(gout.at[srcv.at[0]], bufs[s], gsems[s]).wait()

    def _sfire(j, s):
        pltpu.async_copy(bufs[s], acc_sh.at[dstv.at[j]], ssems[s], add=True)

    def _swait(s):
        pltpu.make_async_copy(bufs[s], acc_sh.at[dstv.at[0]], ssems[s]).wait()

    def _step(j, s, swait=True):
        if swait:
            _swait(s)
        _gfire(j, s)
        s2 = (s + SLOTS - LEAD) % SLOTS
        _gwait(s2)
        _sfire(j - LEAD, s2)

    for j in range(LEAD):
        _gfire(j, j)
    for j in range(LEAD, SLOTS):
        _step(j, j, swait=False)

    def _main(p, _):
        j0 = SLOTS * p + SLOTS
        for s in range(SLOTS):
            _step(j0 + s, s)
        return 0
    lax.fori_loop(0, NCHUNK // SLOTS - 1, _main, 0)

    for j in range(NCHUNK - LEAD, NCHUNK):
        s2 = j % SLOTS
        _gwait(s2)
        _sfire(j, s2)
    for s in range(SLOTS):
        _swait(s)

    plsc.subcore_barrier()

    for k in range(TPB // CH):
        row0 = sid * TPB + k * CH
        pltpu.sync_copy(gout.at[pl.ds(cid * NP + row0, CH), :], buf0)
        pltpu.sync_copy(acc_sh.at[pl.ds(row0, CH), :], buf1)

        def _fin(q, _):
            for u in range(4):
                r = q * 4 + u
                dsp = _splat(disv, k * CH + r)
                for j in range(4):
                    sl = pl.ds(16 * j, 16)
                    buf0[r, sl] = (buf0[r, sl] + buf1[r, sl]) * dsp + bv[sl]
            return 0
        lax.fori_loop(0, CH // 4, _fin, 0)

        @pl.when(cid == 0)
        def _():
            pltpu.sync_copy(buf0, out0.at[pl.ds(row0, CH), :])

        @pl.when(cid == 1)
        def _():
            pltpu.sync_copy(buf0, out1.at[pl.ds(row0, CH), :])


@functools.cache
def _sc_kernel():
    return pl.kernel(
        _sc_body,
        out_type=(jax.ShapeDtypeStruct((NP, DH), jnp.float32),
                  jax.ShapeDtypeStruct((NP, DH), jnp.float32),
                  jax.ShapeDtypeStruct((2 * NP, DH), jnp.float32)),
        mesh=plsc.VectorSubcoreMesh(core_axis_name="c", subcore_axis_name="s",
                                    num_cores=2, num_subcores=16),
        scratch_types=[
            pltpu.VMEM_SHARED((NP, DH), jnp.float32),
            pltpu.VMEM_SHARED((NP,), jnp.float32),
            pltpu.VMEM((NCHUNK, CH), jnp.int32),
            pltpu.VMEM((NCHUNK, CH), jnp.int32),
        ]
        + [pltpu.VMEM((CH, DH), jnp.float32)] * SLOTS
        + [
            pltpu.VMEM((TPB,), jnp.float32),
            pltpu.VMEM((CH,), jnp.float32),
            pltpu.VMEM((DH,), jnp.float32),
        ]
        + [pltpu.SemaphoreType.DMA] * (2 * SLOTS + 1),

        compiler_params=pltpu.CompilerParams(needs_layout_passes=False,
                                             use_tc_tiling_on_sc=False),
    )


def kernel(x, edge_index, W_mu, b_mu, W_logstd, b_logstd):
    w_cat = jnp.stack([W_mu, W_logstd])
    b_cat = jnp.concatenate([b_mu, b_logstd], axis=0)
    h_pad = _matmul(x, w_cat)

    pad = E_PAD - E
    dummy = N + (jnp.arange(pad, dtype=jnp.int32) % (NP - N))
    src_t = jnp.concatenate([edge_index[0], dummy]).reshape(16, NCHUNK, CH)
    dst_t = jnp.concatenate([edge_index[1], dummy]).reshape(16, NCHUNK, CH)

    out0, out1, _g = _sc_kernel()(h_pad, src_t, dst_t, b_cat)
    return (out0[:N], out1[:N])


# speedup vs baseline: 1.0195x; 1.0195x over previous
import functools

import jax
import jax.numpy as jnp
from jax import lax
from jax.experimental import pallas as pl
from jax.experimental.pallas import tpu as pltpu
from jax.experimental.pallas import tpu_sc as plsc

N = 10000
E = 320000
D = 128
DH = 64
NP = 10240
TPB = NP // 16
CH = 128
NCHUNK = 160
EPT = NCHUNK * CH
E_PAD = 16 * EPT
MM_BLK = 512


def _mm_body(x_ref, w_ref, o_ref):
    o_ref[0] = jnp.dot(x_ref[...], w_ref[0],
                       preferred_element_type=jnp.float32)


def _matmul(x_pad, w_cat):
    return pl.pallas_call(
        _mm_body,
        out_shape=jax.ShapeDtypeStruct((2, NP, DH), jnp.float32),
        grid=(NP // MM_BLK, 2),
        in_specs=[
            pl.BlockSpec((MM_BLK, D), lambda i, j: (i, 0)),
            pl.BlockSpec((1, D, DH), lambda i, j: (j, 0, 0)),
        ],
        out_specs=pl.BlockSpec((1, MM_BLK, DH), lambda i, j: (j, i, 0)),
    )(x_pad, w_cat)


def _splat(vec_ref, i):
    return plsc.load_gather(vec_ref, [jnp.full((16,), i, jnp.int32)])


SLOTS = 6
LEAD = SLOTS // 2


def _sc_body(h_hbm, src_hbm, dst_hbm, b_hbm, out0, out1, gout,
             acc_sh, deg_sh, dstv,
             buf0, buf1, buf2, buf3, buf4, buf5,
             ib0, ib1, ib2, ib3, ib4, ib5,
             disv, onesv, bv,
             semg0, semg1, semg2, semg3, semg4, semg5,
             sems0, sems1, sems2, sems3, sems4, sems5,
             semi0, semi1, semi2, semi3, semi4, semi5, semh):
    cid = lax.axis_index("c")
    sid = lax.axis_index("s")

    bufs = (buf0, buf1, buf2, buf3, buf4, buf5)
    ibs = (ib0, ib1, ib2, ib3, ib4, ib5)
    gsems = (semg0, semg1, semg2, semg3, semg4, semg5)
    ssems = (sems0, sems1, sems2, sems3, sems4, sems5)
    isems = (semi0, semi1, semi2, semi3, semi4, semi5)

    zeros16 = jnp.zeros((16,), jnp.float32)
    ones16 = jnp.ones((16,), jnp.float32)

    pltpu.sync_copy(dst_hbm.at[sid], dstv)
    pltpu.sync_copy(b_hbm.at[pl.ds(cid * DH, DH)], bv)

    def _zrow(r, _):
        for j in range(4):
            buf0[r, pl.ds(16 * j, 16)] = zeros16
        return 0
    lax.fori_loop(0, CH, _zrow, 0)

    def _zvec(k, _):
        disv[pl.ds(k * 16, 16)] = zeros16
        return 0
    lax.fori_loop(0, TPB // 16, _zvec, 0)

    for j in range(CH // 16):
        onesv[pl.ds(16 * j, 16)] = ones16

    for k in range(TPB // CH):
        pltpu.sync_copy(buf0, acc_sh.at[pl.ds(sid * TPB + k * CH, CH), :])
    pltpu.sync_copy(disv, deg_sh.at[pl.ds(sid * TPB, TPB)])

    plsc.subcore_barrier()

    def _hist(p, _):
        for i in range(8):
            pltpu.async_copy(onesv, deg_sh.at[dstv.at[p * 8 + i]], semh,
                             add=True)
        for i in range(8):
            pltpu.make_async_copy(onesv, deg_sh.at[dstv.at[p * 8 + i]],
                                  semh).wait()
        return 0
    lax.fori_loop(0, NCHUNK // 8, _hist, 0)

    plsc.subcore_barrier()

    pltpu.sync_copy(deg_sh.at[pl.ds(sid * TPB, TPB)], disv)

    def _newton(k, _):
        d = disv[pl.ds(k * 16, 16)] + 1.0
        y = 0.5 * (d + 1.0)
        for _i in range(16):
            y = 0.5 * (y + d / y)
        disv[pl.ds(k * 16, 16)] = 1.0 / y
        return 0
    lax.fori_loop(0, TPB // 16, _newton, 0)

    for k in range(TPB // CH):
        row0 = sid * TPB + k * CH
        pltpu.sync_copy(h_hbm.at[cid, pl.ds(row0, CH), :], buf0)

        def _scale(q, _):
            for u in range(4):
                r = q * 4 + u
                dsp = _splat(disv, k * CH + r)
                for j in range(4):
                    sl = pl.ds(16 * j, 16)
                    buf0[r, sl] = buf0[r, sl] * dsp
            return 0
        lax.fori_loop(0, CH // 4, _scale, 0)
        pltpu.sync_copy(buf0, gout.at[pl.ds(cid * NP + row0, CH), :])

    plsc.subcore_barrier()

    def _idx(j, s):
        pltpu.async_copy(src_hbm.at[sid, j], ibs[s], isems[s])

    def _gfire(j, s):
        pltpu.make_async_copy(src_hbm.at[sid, 0], ibs[s], isems[s]).wait()
        for v in range(CH // 16):
            sl = pl.ds(16 * v, 16)
            ibs[s][sl] = ibs[s][sl] + cid * NP
        pltpu.async_copy(gout.at[ibs[s]], bufs[s], gsems[s])

    def _gwait(s):
        pltpu.make_async_copy(gout.at[ib0], bufs[s], gsems[s]).wait()

    def _sfire(j, s):
        pltpu.async_copy(bufs[s], acc_sh.at[dstv.at[j]], ssems[s], add=True)

    def _swait(s):
        pltpu.make_async_copy(bufs[s], acc_sh.at[dstv.at[0]], ssems[s]).wait()

    def _step(j, s, swait, fire_idx=True):
        if swait:
            _swait(s)
        _gfire(j, s)
        s2 = (s + LEAD) % SLOTS
        _gwait(s2)
        _sfire(j - LEAD, s2)
        if fire_idx:
            _idx(j + LEAD, s2)

    for s in range(SLOTS):
        _idx(s, s)
    for j in range(LEAD):
        _gfire(j, j)
    for j in range(LEAD, SLOTS):
        _step(j, j, swait=False)

    def _main(p, _):
        j0 = SLOTS * p + SLOTS
        for s in range(SLOTS):
            _step(j0 + s, s, swait=True)
        return 0
    n_main = (NCHUNK - SLOTS - 4) // SLOTS
    lax.fori_loop(0, n_main, _main, 0)

    base = SLOTS + n_main * SLOTS
    _step(base + 0, (base + 0) % SLOTS, swait=True)
    _step(base + 1, (base + 1) % SLOTS, swait=True, fire_idx=False)
    _step(base + 2, (base + 2) % SLOTS, swait=True, fire_idx=False)
    _step(base + 3, (base + 3) % SLOTS, swait=True, fire_idx=False)
    for j in range(base + 1, base + 4):
        s2 = j % SLOTS
        _gwait(s2)
        _sfire(j, s2)
    for s in range(SLOTS):
        _swait(s)

    plsc.subcore_barrier()

    for k in range(TPB // CH):
        row0 = sid * TPB + k * CH
        pltpu.sync_copy(gout.at[pl.ds(cid * NP + row0, CH), :], buf0)
        pltpu.sync_copy(acc_sh.at[pl.ds(row0, CH), :], buf1)

        def _fin(q, _):
            for u in range(4):
                r = q * 4 + u
                dsp = _splat(disv, k * CH + r)
                for j in range(4):
                    sl = pl.ds(16 * j, 16)
                    buf0[r, sl] = (buf0[r, sl] + buf1[r, sl]) * dsp + bv[sl]
            return 0
        lax.fori_loop(0, CH // 4, _fin, 0)

        @pl.when(cid == 0)
        def _():
            pltpu.sync_copy(buf0, out0.at[pl.ds(row0, CH), :])

        @pl.when(cid == 1)
        def _():
            pltpu.sync_copy(buf0, out1.at[pl.ds(row0, CH), :])


@functools.cache
def _sc_kernel():
    return pl.kernel(
        _sc_body,
        out_type=(jax.ShapeDtypeStruct((NP, DH), jnp.float32),
                  jax.ShapeDtypeStruct((NP, DH), jnp.float32),
                  jax.ShapeDtypeStruct((2 * NP, DH), jnp.float32)),
        mesh=plsc.VectorSubcoreMesh(core_axis_name="c", subcore_axis_name="s",
                                    num_cores=2, num_subcores=16),
        scratch_types=[
            pltpu.VMEM_SHARED((NP, DH), jnp.float32),
            pltpu.VMEM_SHARED((NP,), jnp.float32),
            pltpu.VMEM((NCHUNK, CH), jnp.int32),
        ]
        + [pltpu.VMEM((CH, DH), jnp.float32)] * SLOTS
        + [pltpu.VMEM((CH,), jnp.int32)] * SLOTS
        + [
            pltpu.VMEM((TPB,), jnp.float32),
            pltpu.VMEM((CH,), jnp.float32),
            pltpu.VMEM((DH,), jnp.float32),
        ]
        + [pltpu.SemaphoreType.DMA] * (3 * SLOTS + 1),

        compiler_params=pltpu.CompilerParams(needs_layout_passes=False,
                                             use_tc_tiling_on_sc=False),
    )


def kernel(x, edge_index, W_mu, b_mu, W_logstd, b_logstd):
    w_cat = jnp.stack([W_mu, W_logstd])
    b_cat = jnp.concatenate([b_mu, b_logstd], axis=0)
    h_pad = _matmul(x, w_cat)

    pad = E_PAD - E
    dummy = N + (jnp.arange(pad, dtype=jnp.int32) % (NP - N))
    src_t = jnp.concatenate([edge_index[0], dummy]).reshape(16, NCHUNK, CH)
    dst_t = jnp.concatenate([edge_index[1], dummy]).reshape(16, NCHUNK, CH)

    out0, out1, _g = _sc_kernel()(h_pad, src_t, dst_t, b_cat)
    return (out0[:N], out1[:N])
